# baseline (speedup 1.0000x reference)
import functools

import jax
import jax.numpy as jnp
from jax import lax
from jax.experimental import pallas as pl
from jax.experimental.pallas import tpu as pltpu
from jax.experimental.pallas import tpu_sc as plsc

F32 = jnp.float32
V = 10000
VP = 10112
E = 320000
HID = 128
GD = 16
AD = 16
W = 128


def _sc_mesh():
    return plsc.VectorSubcoreMesh(core_axis_name="c", subcore_axis_name="s")


_SC_PARAMS = pltpu.CompilerParams(use_tc_tiling_on_sc=False)
_TC_PARAMS = pltpu.CompilerParams(dimension_semantics=("parallel",))


def _silu(t):
    return t * jax.nn.sigmoid(t)


def sc_gather_hg(h_tbl, g_tbl, idx):
    n = idx.shape[1]
    info = plsc.get_sparse_core_info()
    rps = VP // info.num_subcores

    @functools.partial(
        pl.kernel,
        out_type=[jax.ShapeDtypeStruct((n, HID), F32),
                  jax.ShapeDtypeStruct((n, GD), F32)],
        mesh=_sc_mesh(),
        compiler_params=_SC_PARAMS,
        scratch_types=[pltpu.VMEM_SHARED((VP, HID), F32),
                       pltpu.VMEM_SHARED((VP, GD), F32)],
    )
    def k(h_hbm, g_hbm, i_hbm, oh_hbm, og_hbm, h_s, g_s):
        sid = lax.axis_index("s")
        r0 = sid * rps
        pltpu.sync_copy(h_hbm.at[pl.ds(r0, rps)], h_s.at[pl.ds(r0, rps)])
        pltpu.sync_copy(g_hbm.at[pl.ds(r0, rps)], g_s.at[pl.ds(r0, rps)])
        plsc.subcore_barrier()

        def body(i_v, oh_v, og_v):
            pltpu.sync_copy(h_s.at[i_v.at[0]], oh_v)
            pltpu.sync_copy(g_s.at[i_v.at[0]], og_v)

        pltpu.emit_pipeline(
            body,
            grid=(n // W,),
            in_specs=[pl.BlockSpec((1, W), lambda i: (0, i))],
            out_specs=[pl.BlockSpec((W, HID), lambda i: (i, 0)),
                       pl.BlockSpec((W, GD), lambda i: (i, 0))],
            core_axis_name=("c", "s"),
            dimension_semantics=(pltpu.PARALLEL,),
        )(i_hbm, oh_hbm, og_hbm)

    return k(h_tbl, g_tbl, idx)


def sc_gather_one(table, idx):
    nv, D = table.shape
    n = idx.shape[1]
    info = plsc.get_sparse_core_info()
    rps = nv // info.num_subcores

    @functools.partial(
        pl.kernel,
        out_type=jax.ShapeDtypeStruct((n, D), F32),
        mesh=_sc_mesh(),
        compiler_params=_SC_PARAMS,
        scratch_types=[pltpu.VMEM_SHARED((nv, D), F32)],
    )
    def k(tbl_hbm, i_hbm, o_hbm, tbl_s):
        sid = lax.axis_index("s")
        r0 = sid * rps
        pltpu.sync_copy(tbl_hbm.at[pl.ds(r0, rps)], tbl_s.at[pl.ds(r0, rps)])
        plsc.subcore_barrier()

        def body(i_v, o_v):
            pltpu.sync_copy(tbl_s.at[i_v.at[0]], o_v)

        pltpu.emit_pipeline(
            body,
            grid=(n // W,),
            in_specs=[pl.BlockSpec((1, W), lambda i: (0, i))],
            out_specs=[pl.BlockSpec((W, D), lambda i: (i, 0))],
            core_axis_name=("c", "s"),
            dimension_semantics=(pltpu.PARALLEL,),
        )(i_hbm, o_hbm)

    return k(table, idx)


def sc_scatter_add1(src, idx, zeros_tbl):
    n, D = src.shape
    nv = zeros_tbl.shape[0]
    info = plsc.get_sparse_core_info()
    nc, ns = info.num_cores, info.num_subcores
    rps = nv // ns

    @functools.partial(
        pl.kernel,
        out_type=jax.ShapeDtypeStruct((nc, nv, D), F32),
        mesh=_sc_mesh(),
        compiler_params=_SC_PARAMS,
        scratch_types=[pltpu.VMEM_SHARED((nv, D), F32)],
    )
    def k(s_hbm, i_hbm, z_hbm, o_hbm, acc):
        cid = lax.axis_index("c")
        sid = lax.axis_index("s")
        r0 = sid * rps
        pltpu.sync_copy(z_hbm.at[pl.ds(r0, rps)], acc.at[pl.ds(r0, rps)])
        plsc.subcore_barrier()

        def body(s_v, i_v):
            pltpu.sync_copy(s_v, acc.at[i_v.at[0]], add=True)

        pltpu.emit_pipeline(
            body,
            grid=(n // W,),
            in_specs=[
                pl.BlockSpec((W, D), lambda i: (i, 0)),
                pl.BlockSpec((1, W), lambda i: (0, i)),
            ],
            out_specs=[],
            core_axis_name=("c", "s"),
            dimension_semantics=(pltpu.PARALLEL,),
        )(s_hbm, i_hbm)
        plsc.subcore_barrier()
        pltpu.sync_copy(acc.at[pl.ds(r0, rps)], o_hbm.at[cid, pl.ds(r0, rps)])

    return k(src, idx, zeros_tbl)


def sc_scatter_add2(src_m, src_c, idx, zeros_m, zeros_c):
    n = src_m.shape[0]
    info = plsc.get_sparse_core_info()
    nc, ns = info.num_cores, info.num_subcores
    rps = VP // ns

    @functools.partial(
        pl.kernel,
        out_type=[jax.ShapeDtypeStruct((nc, VP, HID), F32),
                  jax.ShapeDtypeStruct((nc, VP, AD), F32)],
        mesh=_sc_mesh(),
        compiler_params=_SC_PARAMS,
        scratch_types=[pltpu.VMEM_SHARED((VP, HID), F32),
                       pltpu.VMEM_SHARED((VP, AD), F32)],
    )
    def k(sm_hbm, sc_hbm, i_hbm, zm_hbm, zc_hbm, om_hbm, oc_hbm, accm, accc):
        cid = lax.axis_index("c")
        sid = lax.axis_index("s")
        r0 = sid * rps
        pltpu.sync_copy(zm_hbm.at[pl.ds(r0, rps)], accm.at[pl.ds(r0, rps)])
        pltpu.sync_copy(zc_hbm.at[pl.ds(r0, rps)], accc.at[pl.ds(r0, rps)])
        plsc.subcore_barrier()

        def body(sm_v, sc_v, i_v):
            pltpu.sync_copy(sm_v, accm.at[i_v.at[0]], add=True)
            pltpu.sync_copy(sc_v, accc.at[i_v.at[0]], add=True)

        pltpu.emit_pipeline(
            body,
            grid=(n // W,),
            in_specs=[
                pl.BlockSpec((W, HID), lambda i: (i, 0)),
                pl.BlockSpec((W, AD), lambda i: (i, 0)),
                pl.BlockSpec((1, W), lambda i: (0, i)),
            ],
            out_specs=[],
            core_axis_name=("c", "s"),
            dimension_semantics=(pltpu.PARALLEL,),
        )(sm_hbm, sc_hbm, i_hbm)
        plsc.subcore_barrier()
        pltpu.sync_copy(accm.at[pl.ds(r0, rps)], om_hbm.at[cid, pl.ds(r0, rps)])
        pltpu.sync_copy(accc.at[pl.ds(r0, rps)], oc_hbm.at[cid, pl.ds(r0, rps)])

    return k(src_m, src_c, idx, zeros_m, zeros_c)


def tc_edge1(H2, G2, We1h, We1c, we1d, be1, We2, be2, Wa1h, Wa1c, wa1d,
             ba1, Wa2, ba2):
    BE = 2000
    nb = E // BE
    full = lambda a: pl.BlockSpec(a.shape, lambda i: (0,) * a.ndim)

    def body(hr_r, hc_r, gr_r, gc_r, we1h, we1c, we1d_, be1_, we2, be2_,
             wa1h, wa1c, wa1d_, ba1_, wa2, ba2_, m_o, au_o, p8_o):
        hr = hr_r[...]
        hc = hc_r[...]
        xr = gr_r[:, 0:3]
        xc = gc_r[:, 0:3]
        cr = gr_r[:, 3:7]
        diff = xr - xc
        dist_sq = jnp.sum(diff * diff, axis=1, keepdims=True)
        t = (jnp.dot(hr, we1h[...], preferred_element_type=F32)
             + jnp.dot(hc, we1c[...], preferred_element_type=F32)
             + dist_sq * we1d_[...] + be1_[...])
        m1 = _silu(t)
        t2 = jnp.dot(m1, we2[...], preferred_element_type=F32) + be2_[...]
        m_o[...] = _silu(t2)
        norm = jnp.sqrt(dist_sq)
        distf = jnp.maximum(norm, 1e-6)
        unit = (xc - xr) / distf
        ta = (jnp.dot(hr, wa1h[...], preferred_element_type=F32)
              + jnp.dot(cr, wa1c[...], preferred_element_type=F32)
              + distf * wa1d_[...] + ba1_[...])
        alpha = jnp.dot(_silu(ta), wa2[...], preferred_element_type=F32) + ba2_[...]
        au = alpha * unit
        au_o[...] = jnp.concatenate([au, jnp.zeros((BE, AD - 3), F32)], axis=1)
        p8_o[...] = jnp.concatenate([unit, norm, cr], axis=1)

    return pl.pallas_call(
        body,
        grid=(nb,),
        compiler_params=_TC_PARAMS,
        in_specs=[
            pl.BlockSpec((BE, HID), lambda i: (i, 0)),
            pl.BlockSpec((BE, HID), lambda i: (i + nb, 0)),
            pl.BlockSpec((BE, GD), lambda i: (i, 0)),
            pl.BlockSpec((BE, GD), lambda i: (i + nb, 0)),
            full(We1h), full(We1c), full(we1d), full(be1), full(We2),
            full(be2), full(Wa1h), full(Wa1c), full(wa1d), full(ba1),
            full(Wa2), full(ba2),
        ],
        out_specs=[
            pl.BlockSpec((BE, HID), lambda i: (i, 0)),
            pl.BlockSpec((BE, AD), lambda i: (i, 0)),
            pl.BlockSpec((BE, 8), lambda i: (i, 0)),
        ],
        out_shape=[
            jax.ShapeDtypeStruct((E, HID), F32),
            jax.ShapeDtypeStruct((E, AD), F32),
            jax.ShapeDtypeStruct((E, 8), F32),
        ],
    )(H2, H2, G2, G2, We1h, We1c, we1d, be1, We2, be2, Wa1h, Wa1c, wa1d,
      ba1, Wa2, ba2)


def tc_dnorm(dpart):
    def body(dp, dt_o):
        raw = jnp.sum(dp[:, :, 0:3], axis=0)
        nrm = jnp.sqrt(jnp.sum(raw * raw, axis=1, keepdims=True))
        d = raw / jnp.maximum(nrm, 1e-6)
        dt_o[...] = jnp.concatenate([d, jnp.zeros((VP, AD - 3), F32)], axis=1)

    return pl.pallas_call(
        body,
        out_shape=jax.ShapeDtypeStruct((VP, AD), F32),
    )(dpart)


def tc_edge2(m, p8, Dr, ww1a, Ww1c, bw1, Ww2, bw2, Wc1, bc1, Wc2):
    BE = 2000
    nb = E // BE
    full = lambda a: pl.BlockSpec(a.shape, lambda i: (0,) * a.ndim)

    def body(m_r, p8_r, dr_r, ww1a_, ww1c, bw1_, ww2, bw2_, wc1, bc1_, wc2,
             mw_o, c_o):
        unit = p8_r[:, 0:3]
        norm = p8_r[:, 3:4]
        cr = p8_r[:, 4:8]
        d_r = dr_r[:, 0:3]
        align = jnp.abs(jnp.sum(unit * d_r, axis=1, keepdims=True))
        tw = (align * ww1a_[...]
              + jnp.dot(cr, ww1c[...], preferred_element_type=F32) + bw1_[...])
        w = jax.nn.sigmoid(
            jnp.dot(_silu(tw), ww2[...], preferred_element_type=F32) + bw2_[...]
        ) + 0.5
        m_w = m_r[...] * w
        mw_o[...] = m_w
        tc1 = jnp.dot(m_w, wc1[...], preferred_element_type=F32) + bc1_[...]
        coordw = jnp.tanh(jnp.dot(_silu(tc1), wc2[...],
                                  preferred_element_type=F32))
        distf = jnp.maximum(norm, 1e-6)
        contrib = coordw * (-(unit * distf)) / (norm + 1.0)
        c_o[...] = jnp.concatenate(
            [contrib, jnp.zeros((BE, AD - 3), F32)], axis=1)

    return pl.pallas_call(
        body,
        grid=(nb,),
        compiler_params=_TC_PARAMS,
        in_specs=[
            pl.BlockSpec((BE, HID), lambda i: (i, 0)),
            pl.BlockSpec((BE, 8), lambda i: (i, 0)),
            pl.BlockSpec((BE, AD), lambda i: (i, 0)),
            full(ww1a), full(Ww1c), full(bw1), full(Ww2), full(bw2),
            full(Wc1), full(bc1), full(Wc2),
        ],
        out_specs=[
            pl.BlockSpec((BE, HID), lambda i: (i, 0)),
            pl.BlockSpec((BE, AD), lambda i: (i, 0)),
        ],
        out_shape=[
            jax.ShapeDtypeStruct((E, HID), F32),
            jax.ShapeDtypeStruct((E, AD), F32),
        ],
    )(m, p8, Dr, ww1a, Ww1c, bw1, Ww2, bw2, Wc1, bc1, Wc2)


def tc_node(mpart, cpart, h, x, Wn1h, Wn1m, bn1, Wn2, bn2, lng, lnb):
    BV = 2000
    nc = mpart.shape[0]
    full = lambda a: pl.BlockSpec(a.shape, lambda i: (0,) * a.ndim)

    def body(mp, cp, h_r, x_r, wn1h, wn1m, bn1_, wn2, bn2_, lng_, lnb_,
             ho_o, xn_o):
        msg = jnp.sum(mp[...], axis=0)
        coord = jnp.sum(cp[:, :, 0:3], axis=0)
        xn_o[...] = x_r[...] + coord
        t1 = (jnp.dot(h_r[...], wn1h[...], preferred_element_type=F32)
              + jnp.dot(msg, wn1m[...], preferred_element_type=F32)
              + bn1_[...])
        hmid = jnp.dot(_silu(t1), wn2[...], preferred_element_type=F32) + bn2_[...]
        y = h_r[...] + hmid
        mu = jnp.mean(y, axis=1, keepdims=True)
        var = jnp.mean((y - mu) ** 2, axis=1, keepdims=True)
        ho_o[...] = (y - mu) * lax.rsqrt(var + 1e-5) * lng_[...] + lnb_[...]

    return pl.pallas_call(
        body,
        grid=(V // BV,),
        compiler_params=_TC_PARAMS,
        in_specs=[
            pl.BlockSpec((nc, BV, HID), lambda i: (0, i, 0)),
            pl.BlockSpec((nc, BV, AD), lambda i: (0, i, 0)),
            pl.BlockSpec((BV, HID), lambda i: (i, 0)),
            pl.BlockSpec((BV, 3), lambda i: (i, 0)),
            full(Wn1h), full(Wn1m), full(bn1), full(Wn2), full(bn2),
            full(lng), full(lnb),
        ],
        out_specs=[
            pl.BlockSpec((BV, HID), lambda i: (i, 0)),
            pl.BlockSpec((BV, 3), lambda i: (i, 0)),
        ],
        out_shape=[
            jax.ShapeDtypeStruct((V, HID), F32),
            jax.ShapeDtypeStruct((V, 3), F32),
        ],
    )(mpart, cpart, h, x, Wn1h, Wn1m, bn1, Wn2, bn2, lng, lnb)


def kernel(h, x, curvature, edge_index, W_e1, b_e1, W_e2, b_e2, W_c1, b_c1,
           W_c2, W_n1, b_n1, W_n2, b_n2, ln_g, ln_b, W_a1, b_a1, W_a2, b_a2,
           W_w1, b_w1, W_w2, b_w2):
    row = edge_index[0].reshape(1, E)
    col = edge_index[1].reshape(1, E)
    rc = jnp.concatenate([row, col], axis=1)
    h_tbl = jnp.concatenate([h, jnp.zeros((VP - V, HID), F32)], axis=0)
    g_tbl = jnp.concatenate(
        [jnp.concatenate([x, curvature, jnp.zeros((V, GD - 7), F32)], axis=1),
         jnp.zeros((VP - V, GD), F32)], axis=0)

    H2, G2 = sc_gather_hg(h_tbl, g_tbl, rc)

    r2 = lambda a: a.reshape(1, -1)
    m, au16, p8 = tc_edge1(
        H2, G2,
        W_e1[0:HID], W_e1[HID:2 * HID], r2(W_e1[2 * HID]), r2(b_e1),
        W_e2, r2(b_e2),
        W_a1[0:HID], W_a1[HID:HID + 4], r2(W_a1[HID + 4]), r2(b_a1),
        W_a2, r2(b_a2),
    )

    dpart = sc_scatter_add1(au16, row, jnp.zeros((VP, AD), F32))
    dt = tc_dnorm(dpart)
    Dr = sc_gather_one(dt, row)

    m_w, c16 = tc_edge2(
        m, p8, Dr,
        r2(W_w1[0]), W_w1[1:5], r2(b_w1), W_w2, r2(b_w2),
        W_c1, r2(b_c1), W_c2,
    )

    mpart, cpart = sc_scatter_add2(
        m_w, c16, col, jnp.zeros((VP, HID), F32), jnp.zeros((VP, AD), F32))
    h_out, x_new = tc_node(
        mpart, cpart, h, x,
        W_n1[0:HID], W_n1[HID:2 * HID], r2(b_n1), W_n2, r2(b_n2),
        r2(ln_g), r2(ln_b),
    )
    return (h_out, x_new)


# speedup vs baseline: 1.0200x; 1.0200x over previous
import functools

import jax
import jax.numpy as jnp
from jax import lax
from jax.experimental import pallas as pl
from jax.experimental.pallas import tpu as pltpu
from jax.experimental.pallas import tpu_sc as plsc

F32 = jnp.float32
V = 10000
VP = 10112
E = 320000
HID = 128
GD = 16
AD = 16
W = 128


def _sc_mesh():
    return plsc.VectorSubcoreMesh(core_axis_name="c", subcore_axis_name="s")


_SC_PARAMS = pltpu.CompilerParams(use_tc_tiling_on_sc=False)
_TC_PARAMS = pltpu.CompilerParams(dimension_semantics=("parallel",))


def _silu(t):
    return t * jax.nn.sigmoid(t)


def sc_gather_hg(h_tbl, g_tbl, idx):
    n = idx.shape[1]
    info = plsc.get_sparse_core_info()
    rps = VP // info.num_subcores

    @functools.partial(
        pl.kernel,
        out_type=[jax.ShapeDtypeStruct((n, HID), F32),
                  jax.ShapeDtypeStruct((n, GD), F32)],
        mesh=_sc_mesh(),
        compiler_params=_SC_PARAMS,
        scratch_types=[pltpu.VMEM_SHARED((VP, HID), F32),
                       pltpu.VMEM_SHARED((VP, GD), F32)],
    )
    def k(h_hbm, g_hbm, i_hbm, oh_hbm, og_hbm, h_s, g_s):
        sid = lax.axis_index("s")
        r0 = sid * rps
        pltpu.sync_copy(h_hbm.at[pl.ds(r0, rps)], h_s.at[pl.ds(r0, rps)])
        pltpu.sync_copy(g_hbm.at[pl.ds(r0, rps)], g_s.at[pl.ds(r0, rps)])
        plsc.subcore_barrier()

        def body(i_v, oh_v, og_v):
            pltpu.sync_copy(h_s.at[i_v.at[0]], oh_v)
            pltpu.sync_copy(g_s.at[i_v.at[0]], og_v)

        pltpu.emit_pipeline(
            body,
            grid=(n // W,),
            in_specs=[pl.BlockSpec((1, W), lambda i: (0, i))],
            out_specs=[pl.BlockSpec((W, HID), lambda i: (i, 0)),
                       pl.BlockSpec((W, GD), lambda i: (i, 0))],
            core_axis_name=("c", "s"),
            dimension_semantics=(pltpu.PARALLEL,),
        )(i_hbm, oh_hbm, og_hbm)

    return k(h_tbl, g_tbl, idx)


def sc_gather_one(table, idx):
    nv, D = table.shape
    n = idx.shape[1]
    info = plsc.get_sparse_core_info()
    rps = nv // info.num_subcores

    @functools.partial(
        pl.kernel,
        out_type=jax.ShapeDtypeStruct((n, D), F32),
        mesh=_sc_mesh(),
        compiler_params=_SC_PARAMS,
        scratch_types=[pltpu.VMEM_SHARED((nv, D), F32)],
    )
    def k(tbl_hbm, i_hbm, o_hbm, tbl_s):
        sid = lax.axis_index("s")
        r0 = sid * rps
        pltpu.sync_copy(tbl_hbm.at[pl.ds(r0, rps)], tbl_s.at[pl.ds(r0, rps)])
        plsc.subcore_barrier()

        def body(i_v, o_v):
            pltpu.sync_copy(tbl_s.at[i_v.at[0]], o_v)

        pltpu.emit_pipeline(
            body,
            grid=(n // W,),
            in_specs=[pl.BlockSpec((1, W), lambda i: (0, i))],
            out_specs=[pl.BlockSpec((W, D), lambda i: (i, 0))],
            core_axis_name=("c", "s"),
            dimension_semantics=(pltpu.PARALLEL,),
        )(i_hbm, o_hbm)

    return k(table, idx)


def sc_scatter_add1(src, idx, zeros_tbl):
    n, D = src.shape
    nv = zeros_tbl.shape[0]
    info = plsc.get_sparse_core_info()
    nc, ns = info.num_cores, info.num_subcores
    rps = nv // ns

    @functools.partial(
        pl.kernel,
        out_type=jax.ShapeDtypeStruct((nc, nv, D), F32),
        mesh=_sc_mesh(),
        compiler_params=_SC_PARAMS,
        scratch_types=[pltpu.VMEM_SHARED((nv, D), F32)],
    )
    def k(s_hbm, i_hbm, z_hbm, o_hbm, acc):
        cid = lax.axis_index("c")
        sid = lax.axis_index("s")
        r0 = sid * rps
        pltpu.sync_copy(z_hbm.at[pl.ds(r0, rps)], acc.at[pl.ds(r0, rps)])
        plsc.subcore_barrier()

        def body(s_v, i_v):
            pltpu.sync_copy(s_v, acc.at[i_v.at[0]], add=True)

        pltpu.emit_pipeline(
            body,
            grid=(n // W,),
            in_specs=[
                pl.BlockSpec((W, D), lambda i: (i, 0)),
                pl.BlockSpec((1, W), lambda i: (0, i)),
            ],
            out_specs=[],
            core_axis_name=("c", "s"),
            dimension_semantics=(pltpu.PARALLEL,),
        )(s_hbm, i_hbm)
        plsc.subcore_barrier()
        pltpu.sync_copy(acc.at[pl.ds(r0, rps)], o_hbm.at[cid, pl.ds(r0, rps)])

    return k(src, idx, zeros_tbl)


def sc_scatter_add2(src_m, src_c, idx, zeros_m, zeros_c):
    n = src_m.shape[0]
    info = plsc.get_sparse_core_info()
    nc, ns = info.num_cores, info.num_subcores
    rps = VP // ns

    @functools.partial(
        pl.kernel,
        out_type=[jax.ShapeDtypeStruct((nc, VP, HID), F32),
                  jax.ShapeDtypeStruct((nc, VP, AD), F32)],
        mesh=_sc_mesh(),
        compiler_params=_SC_PARAMS,
        scratch_types=[pltpu.VMEM_SHARED((VP, HID), F32),
                       pltpu.VMEM_SHARED((VP, AD), F32)],
    )
    def k(sm_hbm, sc_hbm, i_hbm, zm_hbm, zc_hbm, om_hbm, oc_hbm, accm, accc):
        cid = lax.axis_index("c")
        sid = lax.axis_index("s")
        r0 = sid * rps
        pltpu.sync_copy(zm_hbm.at[pl.ds(r0, rps)], accm.at[pl.ds(r0, rps)])
        pltpu.sync_copy(zc_hbm.at[pl.ds(r0, rps)], accc.at[pl.ds(r0, rps)])
        plsc.subcore_barrier()

        def body(sm_v, sc_v, i_v):
            pltpu.sync_copy(sm_v, accm.at[i_v.at[0]], add=True)
            pltpu.sync_copy(sc_v, accc.at[i_v.at[0]], add=True)

        pltpu.emit_pipeline(
            body,
            grid=(n // W,),
            in_specs=[
                pl.BlockSpec((W, HID), lambda i: (i, 0)),
                pl.BlockSpec((W, AD), lambda i: (i, 0)),
                pl.BlockSpec((1, W), lambda i: (0, i)),
            ],
            out_specs=[],
            core_axis_name=("c", "s"),
            dimension_semantics=(pltpu.PARALLEL,),
        )(sm_hbm, sc_hbm, i_hbm)
        plsc.subcore_barrier()
        pltpu.sync_copy(accm.at[pl.ds(r0, rps)], om_hbm.at[cid, pl.ds(r0, rps)])
        pltpu.sync_copy(accc.at[pl.ds(r0, rps)], oc_hbm.at[cid, pl.ds(r0, rps)])

    return k(src_m, src_c, idx, zeros_m, zeros_c)


def tc_edge1(H2, G2, We1h, We1c, we1d, be1, We2, be2, Wa1h, Wa1c, wa1d,
             ba1, Wa2, ba2):
    BE = 2000
    ne = H2.shape[0] // 2
    nb = ne // BE
    full = lambda a: pl.BlockSpec(a.shape, lambda i: (0,) * a.ndim)

    def body(hr_r, hc_r, gr_r, gc_r, we1h, we1c, we1d_, be1_, we2, be2_,
             wa1h, wa1c, wa1d_, ba1_, wa2, ba2_, m_o, au_o, p8_o):
        hr = hr_r[...]
        hc = hc_r[...]
        xr = gr_r[:, 0:3]
        xc = gc_r[:, 0:3]
        cr = gr_r[:, 3:7]
        diff = xr - xc
        dist_sq = jnp.sum(diff * diff, axis=1, keepdims=True)
        t = (jnp.dot(hr, we1h[...], preferred_element_type=F32)
             + jnp.dot(hc, we1c[...], preferred_element_type=F32)
             + dist_sq * we1d_[...] + be1_[...])
        m1 = _silu(t)
        t2 = jnp.dot(m1, we2[...], preferred_element_type=F32) + be2_[...]
        m_o[...] = _silu(t2)
        norm = jnp.sqrt(dist_sq)
        distf = jnp.maximum(norm, 1e-6)
        unit = (xc - xr) / distf
        ta = (jnp.dot(hr, wa1h[...], preferred_element_type=F32)
              + jnp.dot(cr, wa1c[...], preferred_element_type=F32)
              + distf * wa1d_[...] + ba1_[...])
        alpha = jnp.dot(_silu(ta), wa2[...], preferred_element_type=F32) + ba2_[...]
        au = alpha * unit
        au_o[...] = jnp.concatenate([au, jnp.zeros((BE, AD - 3), F32)], axis=1)
        p8_o[...] = jnp.concatenate([unit, norm, cr], axis=1)

    return pl.pallas_call(
        body,
        grid=(nb,),
        compiler_params=_TC_PARAMS,
        in_specs=[
            pl.BlockSpec((BE, HID), lambda i: (i, 0)),
            pl.BlockSpec((BE, HID), lambda i: (i + nb, 0)),
            pl.BlockSpec((BE, GD), lambda i: (i, 0)),
            pl.BlockSpec((BE, GD), lambda i: (i + nb, 0)),
            full(We1h), full(We1c), full(we1d), full(be1), full(We2),
            full(be2), full(Wa1h), full(Wa1c), full(wa1d), full(ba1),
            full(Wa2), full(ba2),
        ],
        out_specs=[
            pl.BlockSpec((BE, HID), lambda i: (i, 0)),
            pl.BlockSpec((BE, AD), lambda i: (i, 0)),
            pl.BlockSpec((BE, 8), lambda i: (i, 0)),
        ],
        out_shape=[
            jax.ShapeDtypeStruct((ne, HID), F32),
            jax.ShapeDtypeStruct((ne, AD), F32),
            jax.ShapeDtypeStruct((ne, 8), F32),
        ],
    )(H2, H2, G2, G2, We1h, We1c, we1d, be1, We2, be2, Wa1h, Wa1c, wa1d,
      ba1, Wa2, ba2)


def tc_dnorm(dpart):
    def body(dp, dt_o):
        raw = jnp.sum(dp[:, :, 0:3], axis=0)
        nrm = jnp.sqrt(jnp.sum(raw * raw, axis=1, keepdims=True))
        d = raw / jnp.maximum(nrm, 1e-6)
        dt_o[...] = jnp.concatenate([d, jnp.zeros((VP, AD - 3), F32)], axis=1)

    return pl.pallas_call(
        body,
        out_shape=jax.ShapeDtypeStruct((VP, AD), F32),
    )(dpart)


def tc_edge2(m, p8, Dr, ww1a, Ww1c, bw1, Ww2, bw2, Wc1, bc1, Wc2):
    BE = 2000
    ne = m.shape[0]
    nb = ne // BE
    full = lambda a: pl.BlockSpec(a.shape, lambda i: (0,) * a.ndim)

    def body(m_r, p8_r, dr_r, ww1a_, ww1c, bw1_, ww2, bw2_, wc1, bc1_, wc2,
             mw_o, c_o):
        unit = p8_r[:, 0:3]
        norm = p8_r[:, 3:4]
        cr = p8_r[:, 4:8]
        d_r = dr_r[:, 0:3]
        align = jnp.abs(jnp.sum(unit * d_r, axis=1, keepdims=True))
        tw = (align * ww1a_[...]
              + jnp.dot(cr, ww1c[...], preferred_element_type=F32) + bw1_[...])
        w = jax.nn.sigmoid(
            jnp.dot(_silu(tw), ww2[...], preferred_element_type=F32) + bw2_[...]
        ) + 0.5
        m_w = m_r[...] * w
        mw_o[...] = m_w
        tc1 = jnp.dot(m_w, wc1[...], preferred_element_type=F32) + bc1_[...]
        coordw = jnp.tanh(jnp.dot(_silu(tc1), wc2[...],
                                  preferred_element_type=F32))
        distf = jnp.maximum(norm, 1e-6)
        contrib = coordw * (-(unit * distf)) / (norm + 1.0)
        c_o[...] = jnp.concatenate(
            [contrib, jnp.zeros((BE, AD - 3), F32)], axis=1)

    return pl.pallas_call(
        body,
        grid=(nb,),
        compiler_params=_TC_PARAMS,
        in_specs=[
            pl.BlockSpec((BE, HID), lambda i: (i, 0)),
            pl.BlockSpec((BE, 8), lambda i: (i, 0)),
            pl.BlockSpec((BE, AD), lambda i: (i, 0)),
            full(ww1a), full(Ww1c), full(bw1), full(Ww2), full(bw2),
            full(Wc1), full(bc1), full(Wc2),
        ],
        out_specs=[
            pl.BlockSpec((BE, HID), lambda i: (i, 0)),
            pl.BlockSpec((BE, AD), lambda i: (i, 0)),
        ],
        out_shape=[
            jax.ShapeDtypeStruct((ne, HID), F32),
            jax.ShapeDtypeStruct((ne, AD), F32),
        ],
    )(m, p8, Dr, ww1a, Ww1c, bw1, Ww2, bw2, Wc1, bc1, Wc2)


def tc_node(mpart, cpart, h, x, Wn1h, Wn1m, bn1, Wn2, bn2, lng, lnb):
    BV = 2000
    nc = mpart.shape[0]
    full = lambda a: pl.BlockSpec(a.shape, lambda i: (0,) * a.ndim)

    def body(mp, cp, h_r, x_r, wn1h, wn1m, bn1_, wn2, bn2_, lng_, lnb_,
             ho_o, xn_o):
        msg = jnp.sum(mp[...], axis=0)
        coord = jnp.sum(cp[:, :, 0:3], axis=0)
        xn_o[...] = x_r[...] + coord
        t1 = (jnp.dot(h_r[...], wn1h[...], preferred_element_type=F32)
              + jnp.dot(msg, wn1m[...], preferred_element_type=F32)
              + bn1_[...])
        hmid = jnp.dot(_silu(t1), wn2[...], preferred_element_type=F32) + bn2_[...]
        y = h_r[...] + hmid
        mu = jnp.mean(y, axis=1, keepdims=True)
        var = jnp.mean((y - mu) ** 2, axis=1, keepdims=True)
        ho_o[...] = (y - mu) * lax.rsqrt(var + 1e-5) * lng_[...] + lnb_[...]

    return pl.pallas_call(
        body,
        grid=(V // BV,),
        compiler_params=_TC_PARAMS,
        in_specs=[
            pl.BlockSpec((nc, BV, HID), lambda i: (0, i, 0)),
            pl.BlockSpec((nc, BV, AD), lambda i: (0, i, 0)),
            pl.BlockSpec((BV, HID), lambda i: (i, 0)),
            pl.BlockSpec((BV, 3), lambda i: (i, 0)),
            full(Wn1h), full(Wn1m), full(bn1), full(Wn2), full(bn2),
            full(lng), full(lnb),
        ],
        out_specs=[
            pl.BlockSpec((BV, HID), lambda i: (i, 0)),
            pl.BlockSpec((BV, 3), lambda i: (i, 0)),
        ],
        out_shape=[
            jax.ShapeDtypeStruct((V, HID), F32),
            jax.ShapeDtypeStruct((V, 3), F32),
        ],
    )(mpart, cpart, h, x, Wn1h, Wn1m, bn1, Wn2, bn2, lng, lnb)


NCHUNK = 2
EC = E // NCHUNK


def kernel(h, x, curvature, edge_index, W_e1, b_e1, W_e2, b_e2, W_c1, b_c1,
           W_c2, W_n1, b_n1, W_n2, b_n2, ln_g, ln_b, W_a1, b_a1, W_a2, b_a2,
           W_w1, b_w1, W_w2, b_w2):
    row = edge_index[0].reshape(1, E)
    col = edge_index[1].reshape(1, E)
    h_tbl = jnp.concatenate([h, jnp.zeros((VP - V, HID), F32)], axis=0)
    g_tbl = jnp.concatenate(
        [jnp.concatenate([x, curvature, jnp.zeros((V, GD - 7), F32)], axis=1),
         jnp.zeros((VP - V, GD), F32)], axis=0)

    r2 = lambda a: a.reshape(1, -1)
    rows = [row[:, i * EC:(i + 1) * EC] for i in range(NCHUNK)]
    cols = [col[:, i * EC:(i + 1) * EC] for i in range(NCHUNK)]
    rcs = [jnp.concatenate([rows[i], cols[i]], axis=1) for i in range(NCHUNK)]

    HG = [sc_gather_hg(h_tbl, g_tbl, rcs[i]) for i in range(NCHUNK)]

    e1 = [tc_edge1(
        HG[i][0], HG[i][1],
        W_e1[0:HID], W_e1[HID:2 * HID], r2(W_e1[2 * HID]), r2(b_e1),
        W_e2, r2(b_e2),
        W_a1[0:HID], W_a1[HID:HID + 4], r2(W_a1[HID + 4]), r2(b_a1),
        W_a2, r2(b_a2),
    ) for i in range(NCHUNK)]

    zad = jnp.zeros((VP, AD), F32)
    dparts = [sc_scatter_add1(e1[i][1], rows[i], zad) for i in range(NCHUNK)]
    dt = tc_dnorm(jnp.concatenate(dparts, axis=0))
    Dr = [sc_gather_one(dt, rows[i]) for i in range(NCHUNK)]

    e2 = [tc_edge2(
        e1[i][0], e1[i][2], Dr[i],
        r2(W_w1[0]), W_w1[1:5], r2(b_w1), W_w2, r2(b_w2),
        W_c1, r2(b_c1), W_c2,
    ) for i in range(NCHUNK)]

    zh = jnp.zeros((VP, HID), F32)
    parts = [sc_scatter_add2(e2[i][0], e2[i][1], cols[i], zh, zad)
             for i in range(NCHUNK)]
    mpart = jnp.concatenate([p[0] for p in parts], axis=0)
    cpart = jnp.concatenate([p[1] for p in parts], axis=0)
    h_out, x_new = tc_node(
        mpart, cpart, h, x,
        W_n1[0:HID], W_n1[HID:2 * HID], r2(b_n1), W_n2, r2(b_n2),
        r2(ln_g), r2(ln_b),
    )
    return (h_out, x_new)


# speedup vs baseline: 1.2048x; 1.1811x over previous
import functools

import jax
import jax.numpy as jnp
from jax import lax
from jax.experimental import pallas as pl
from jax.experimental.pallas import tpu as pltpu
from jax.experimental.pallas import tpu_sc as plsc

F32 = jnp.float32
V = 10000
VP = 10112
E = 320000
HID = 128
GD = 16
AD = 16
W = 128


def _sc_mesh():
    return plsc.VectorSubcoreMesh(core_axis_name="c", subcore_axis_name="s")


_SC_PARAMS = pltpu.CompilerParams(use_tc_tiling_on_sc=False)
_TC_PARAMS = pltpu.CompilerParams(dimension_semantics=("parallel",))


def _silu(t):
    return t * jax.nn.sigmoid(t)


def sc_gather_one(table, idx):
    nv, D = table.shape
    n = idx.shape[1]
    info = plsc.get_sparse_core_info()
    rps = nv // info.num_subcores

    @functools.partial(
        pl.kernel,
        out_type=jax.ShapeDtypeStruct((n, D), F32),
        mesh=_sc_mesh(),
        compiler_params=_SC_PARAMS,
        scratch_types=[pltpu.VMEM_SHARED((nv, D), F32)],
    )
    def k(tbl_hbm, i_hbm, o_hbm, tbl_s):
        sid = lax.axis_index("s")
        r0 = sid * rps
        pltpu.sync_copy(tbl_hbm.at[pl.ds(r0, rps)], tbl_s.at[pl.ds(r0, rps)])
        plsc.subcore_barrier()

        def body(i_v, o_v):
            pltpu.sync_copy(tbl_s.at[i_v.at[0]], o_v)

        pltpu.emit_pipeline(
            body,
            grid=(n // W,),
            in_specs=[pl.BlockSpec((1, W), lambda i: (0, i))],
            out_specs=[pl.BlockSpec((W, D), lambda i: (i, 0))],
            core_axis_name=("c", "s"),
            dimension_semantics=(pltpu.PARALLEL,),
        )(i_hbm, o_hbm)

    return k(table, idx)


def sc_scatter_add1(src, idx, zeros_tbl):
    n, D = src.shape
    nv = zeros_tbl.shape[0]
    info = plsc.get_sparse_core_info()
    nc, ns = info.num_cores, info.num_subcores
    rps = nv // ns

    @functools.partial(
        pl.kernel,
        out_type=jax.ShapeDtypeStruct((nc, nv, D), F32),
        mesh=_sc_mesh(),
        compiler_params=_SC_PARAMS,
        scratch_types=[pltpu.VMEM_SHARED((nv, D), F32)],
    )
    def k(s_hbm, i_hbm, z_hbm, o_hbm, acc):
        cid = lax.axis_index("c")
        sid = lax.axis_index("s")
        r0 = sid * rps
        pltpu.sync_copy(z_hbm.at[pl.ds(r0, rps)], acc.at[pl.ds(r0, rps)])
        plsc.subcore_barrier()

        def body(s_v, i_v):
            pltpu.sync_copy(s_v, acc.at[i_v.at[0]], add=True)

        pltpu.emit_pipeline(
            body,
            grid=(n // W,),
            in_specs=[
                pl.BlockSpec((W, D), lambda i: (i, 0)),
                pl.BlockSpec((1, W), lambda i: (0, i)),
            ],
            out_specs=[],
            core_axis_name=("c", "s"),
            dimension_semantics=(pltpu.PARALLEL,),
        )(s_hbm, i_hbm)
        plsc.subcore_barrier()
        pltpu.sync_copy(acc.at[pl.ds(r0, rps)], o_hbm.at[cid, pl.ds(r0, rps)])

    return k(src, idx, zeros_tbl)


def sc_scatter_add2(src_m, src_c, idx, zeros_m, zeros_c):
    n = src_m.shape[0]
    info = plsc.get_sparse_core_info()
    nc, ns = info.num_cores, info.num_subcores
    rps = VP // ns

    @functools.partial(
        pl.kernel,
        out_type=[jax.ShapeDtypeStruct((nc, VP, HID), F32),
                  jax.ShapeDtypeStruct((nc, VP, AD), F32)],
        mesh=_sc_mesh(),
        compiler_params=_SC_PARAMS,
        scratch_types=[pltpu.VMEM_SHARED((VP, HID), F32),
                       pltpu.VMEM_SHARED((VP, AD), F32)],
    )
    def k(sm_hbm, sc_hbm, i_hbm, zm_hbm, zc_hbm, om_hbm, oc_hbm, accm, accc):
        cid = lax.axis_index("c")
        sid = lax.axis_index("s")
        r0 = sid * rps
        pltpu.sync_copy(zm_hbm.at[pl.ds(r0, rps)], accm.at[pl.ds(r0, rps)])
        pltpu.sync_copy(zc_hbm.at[pl.ds(r0, rps)], accc.at[pl.ds(r0, rps)])
        plsc.subcore_barrier()

        def body(sm_v, sc_v, i_v):
            pltpu.sync_copy(sm_v, accm.at[i_v.at[0]], add=True)
            pltpu.sync_copy(sc_v, accc.at[i_v.at[0]], add=True)

        pltpu.emit_pipeline(
            body,
            grid=(n // W,),
            in_specs=[
                pl.BlockSpec((W, HID), lambda i: (i, 0)),
                pl.BlockSpec((W, AD), lambda i: (i, 0)),
                pl.BlockSpec((1, W), lambda i: (0, i)),
            ],
            out_specs=[],
            core_axis_name=("c", "s"),
            dimension_semantics=(pltpu.PARALLEL,),
        )(sm_hbm, sc_hbm, i_hbm)
        plsc.subcore_barrier()
        pltpu.sync_copy(accm.at[pl.ds(r0, rps)], om_hbm.at[cid, pl.ds(r0, rps)])
        pltpu.sync_copy(accc.at[pl.ds(r0, rps)], oc_hbm.at[cid, pl.ds(r0, rps)])

    return k(src_m, src_c, idx, zeros_m, zeros_c)


def tc_edge1(H2, G2, We1h, We1c, we1d, be1, We2, be2, Wa1h, Wa1c, wa1d,
             ba1, Wa2, ba2):
    BE = 3200
    ne = H2.shape[0] // 2
    nb = ne // BE
    full = lambda a: pl.BlockSpec(a.shape, lambda i: (0,) * a.ndim)

    def body(hr_r, hc_r, gr_r, gc_r, we1h, we1c, we1d_, be1_, we2, be2_,
             wa1h, wa1c, wa1d_, ba1_, wa2, ba2_, m_o, au_o, p8_o):
        hr = hr_r[...]
        hc = hc_r[...]
        xr = gr_r[:, 0:3]
        xc = gc_r[:, 0:3]
        cr = gr_r[:, 3:7]
        diff = xr - xc
        dist_sq = jnp.sum(diff * diff, axis=1, keepdims=True)
        t = (jnp.dot(hr, we1h[...], preferred_element_type=F32)
             + jnp.dot(hc, we1c[...], preferred_element_type=F32)
             + dist_sq * we1d_[...] + be1_[...])
        m1 = _silu(t)
        t2 = jnp.dot(m1, we2[...], preferred_element_type=F32) + be2_[...]
        m_o[...] = _silu(t2)
        norm = jnp.sqrt(dist_sq)
        distf = jnp.maximum(norm, 1e-6)
        unit = (xc - xr) / distf
        ta = (jnp.dot(hr, wa1h[...], preferred_element_type=F32)
              + jnp.dot(cr, wa1c[...], preferred_element_type=F32)
              + distf * wa1d_[...] + ba1_[...])
        alpha = jnp.dot(_silu(ta), wa2[...], preferred_element_type=F32) + ba2_[...]
        au = alpha * unit
        au_o[...] = jnp.concatenate(
            [au, jnp.zeros((BE, HID - 3), F32)], axis=1)
        p8_o[...] = jnp.concatenate([unit, norm, cr], axis=1)

    return pl.pallas_call(
        body,
        grid=(nb,),
        compiler_params=_TC_PARAMS,
        in_specs=[
            pl.BlockSpec((BE, HID), lambda i: (i, 0)),
            pl.BlockSpec((BE, HID), lambda i: (i + nb, 0)),
            pl.BlockSpec((BE, HID), lambda i: (i, 0)),
            pl.BlockSpec((BE, HID), lambda i: (i + nb, 0)),
            full(We1h), full(We1c), full(we1d), full(be1), full(We2),
            full(be2), full(Wa1h), full(Wa1c), full(wa1d), full(ba1),
            full(Wa2), full(ba2),
        ],
        out_specs=[
            pl.BlockSpec((BE, HID), lambda i: (i, 0)),
            pl.BlockSpec((BE, HID), lambda i: (i, 0)),
            pl.BlockSpec((BE, 8), lambda i: (i, 0)),
        ],
        out_shape=[
            jax.ShapeDtypeStruct((ne, HID), F32),
            jax.ShapeDtypeStruct((ne, HID), F32),
            jax.ShapeDtypeStruct((ne, 8), F32),
        ],
    )(H2, H2, G2, G2, We1h, We1c, we1d, be1, We2, be2, Wa1h, Wa1c, wa1d,
      ba1, Wa2, ba2)


def tc_dnorm(dpart):
    def body(dp, dt_o):
        raw = jnp.sum(dp[:, :, 0:3], axis=0)
        nrm = jnp.sqrt(jnp.sum(raw * raw, axis=1, keepdims=True))
        d = raw / jnp.maximum(nrm, 1e-6)
        dt_o[...] = jnp.concatenate(
            [d, jnp.zeros((VP, HID - 3), F32)], axis=1)

    return pl.pallas_call(
        body,
        out_shape=jax.ShapeDtypeStruct((VP, HID), F32),
    )(dpart)


def tc_edge2(m, p8, Dr, ww1a, Ww1c, bw1, Ww2, bw2, Wc1, bc1, Wc2):
    BE = 3200
    ne = m.shape[0]
    nb = ne // BE
    full = lambda a: pl.BlockSpec(a.shape, lambda i: (0,) * a.ndim)

    def body(m_r, p8_r, dr_r, ww1a_, ww1c, bw1_, ww2, bw2_, wc1, bc1_, wc2,
             mw_o, c_o):
        unit = p8_r[:, 0:3]
        norm = p8_r[:, 3:4]
        cr = p8_r[:, 4:8]
        d_r = dr_r[:, 0:3]
        align = jnp.abs(jnp.sum(unit * d_r, axis=1, keepdims=True))
        tw = (align * ww1a_[...]
              + jnp.dot(cr, ww1c[...], preferred_element_type=F32) + bw1_[...])
        w = jax.nn.sigmoid(
            jnp.dot(_silu(tw), ww2[...], preferred_element_type=F32) + bw2_[...]
        ) + 0.5
        m_w = m_r[...] * w
        mw_o[...] = m_w
        tc1 = jnp.dot(m_w, wc1[...], preferred_element_type=F32) + bc1_[...]
        coordw = jnp.tanh(jnp.dot(_silu(tc1), wc2[...],
                                  preferred_element_type=F32))
        distf = jnp.maximum(norm, 1e-6)
        contrib = coordw * (-(unit * distf)) / (norm + 1.0)
        c_o[...] = jnp.concatenate(
            [contrib, jnp.zeros((BE, AD - 3), F32)], axis=1)

    return pl.pallas_call(
        body,
        grid=(nb,),
        compiler_params=_TC_PARAMS,
        in_specs=[
            pl.BlockSpec((BE, HID), lambda i: (i, 0)),
            pl.BlockSpec((BE, 8), lambda i: (i, 0)),
            pl.BlockSpec((BE, HID), lambda i: (i, 0)),
            full(ww1a), full(Ww1c), full(bw1), full(Ww2), full(bw2),
            full(Wc1), full(bc1), full(Wc2),
        ],
        out_specs=[
            pl.BlockSpec((BE, HID), lambda i: (i, 0)),
            pl.BlockSpec((BE, AD), lambda i: (i, 0)),
        ],
        out_shape=[
            jax.ShapeDtypeStruct((ne, HID), F32),
            jax.ShapeDtypeStruct((ne, AD), F32),
        ],
    )(m, p8, Dr, ww1a, Ww1c, bw1, Ww2, bw2, Wc1, bc1, Wc2)


def tc_node(mpart, cpart, h, x, Wn1h, Wn1m, bn1, Wn2, bn2, lng, lnb):
    BV = 2000
    nc = mpart.shape[0]
    full = lambda a: pl.BlockSpec(a.shape, lambda i: (0,) * a.ndim)

    def body(mp, cp, h_r, x_r, wn1h, wn1m, bn1_, wn2, bn2_, lng_, lnb_,
             ho_o, xn_o):
        msg = jnp.sum(mp[...], axis=0)
        coord = jnp.sum(cp[:, :, 0:3], axis=0)
        xn_o[...] = x_r[...] + coord
        t1 = (jnp.dot(h_r[...], wn1h[...], preferred_element_type=F32)
              + jnp.dot(msg, wn1m[...], preferred_element_type=F32)
              + bn1_[...])
        hmid = jnp.dot(_silu(t1), wn2[...], preferred_element_type=F32) + bn2_[...]
        y = h_r[...] + hmid
        mu = jnp.mean(y, axis=1, keepdims=True)
        var = jnp.mean((y - mu) ** 2, axis=1, keepdims=True)
        ho_o[...] = (y - mu) * lax.rsqrt(var + 1e-5) * lng_[...] + lnb_[...]

    return pl.pallas_call(
        body,
        grid=(V // BV,),
        compiler_params=_TC_PARAMS,
        in_specs=[
            pl.BlockSpec((nc, BV, HID), lambda i: (0, i, 0)),
            pl.BlockSpec((nc, BV, AD), lambda i: (0, i, 0)),
            pl.BlockSpec((BV, HID), lambda i: (i, 0)),
            pl.BlockSpec((BV, 3), lambda i: (i, 0)),
            full(Wn1h), full(Wn1m), full(bn1), full(Wn2), full(bn2),
            full(lng), full(lnb),
        ],
        out_specs=[
            pl.BlockSpec((BV, HID), lambda i: (i, 0)),
            pl.BlockSpec((BV, 3), lambda i: (i, 0)),
        ],
        out_shape=[
            jax.ShapeDtypeStruct((V, HID), F32),
            jax.ShapeDtypeStruct((V, 3), F32),
        ],
    )(mpart, cpart, h, x, Wn1h, Wn1m, bn1, Wn2, bn2, lng, lnb)


NCHUNK = 2
EC = E // NCHUNK


def kernel(h, x, curvature, edge_index, W_e1, b_e1, W_e2, b_e2, W_c1, b_c1,
           W_c2, W_n1, b_n1, W_n2, b_n2, ln_g, ln_b, W_a1, b_a1, W_a2, b_a2,
           W_w1, b_w1, W_w2, b_w2):
    row = edge_index[0].reshape(1, E)
    col = edge_index[1].reshape(1, E)
    h_tbl = jnp.concatenate([h, jnp.zeros((VP - V, HID), F32)], axis=0)
    g_tbl = jnp.concatenate(
        [jnp.concatenate([x, curvature, jnp.zeros((V, HID - 7), F32)], axis=1),
         jnp.zeros((VP - V, HID), F32)], axis=0)

    r2 = lambda a: a.reshape(1, -1)
    rows = [row[:, i * EC:(i + 1) * EC] for i in range(NCHUNK)]
    cols = [col[:, i * EC:(i + 1) * EC] for i in range(NCHUNK)]
    rcs = [jnp.concatenate([rows[i], cols[i]], axis=1) for i in range(NCHUNK)]

    H2 = [sc_gather_one(h_tbl, rcs[i]) for i in range(NCHUNK)]
    G2 = [sc_gather_one(g_tbl, rcs[i]) for i in range(NCHUNK)]

    e1 = [tc_edge1(
        H2[i], G2[i],
        W_e1[0:HID], W_e1[HID:2 * HID], r2(W_e1[2 * HID]), r2(b_e1),
        W_e2, r2(b_e2),
        W_a1[0:HID], W_a1[HID:HID + 4], r2(W_a1[HID + 4]), r2(b_a1),
        W_a2, r2(b_a2),
    ) for i in range(NCHUNK)]

    zh = jnp.zeros((VP, HID), F32)
    dparts = [sc_scatter_add1(e1[i][1], rows[i], zh) for i in range(NCHUNK)]
    dt = tc_dnorm(jnp.concatenate(dparts, axis=0))
    Dr = [sc_gather_one(dt, rows[i]) for i in range(NCHUNK)]

    e2 = [tc_edge2(
        e1[i][0], e1[i][2], Dr[i],
        r2(W_w1[0]), W_w1[1:5], r2(b_w1), W_w2, r2(b_w2),
        W_c1, r2(b_c1), W_c2,
    ) for i in range(NCHUNK)]

    zad = jnp.zeros((VP, AD), F32)
    parts = [sc_scatter_add2(e2[i][0], e2[i][1], cols[i], zh, zad)
             for i in range(NCHUNK)]
    mpart = jnp.concatenate([p[0] for p in parts], axis=0)
    cpart = jnp.concatenate([p[1] for p in parts], axis=0)
    h_out, x_new = tc_node(
        mpart, cpart, h, x,
        W_n1[0:HID], W_n1[HID:2 * HID], r2(b_n1), W_n2, r2(b_n2),
        r2(ln_g), r2(ln_b),
    )
    return (h_out, x_new)


# speedup vs baseline: 1.5622x; 1.2966x over previous
import functools

import jax
import jax.numpy as jnp
from jax import lax
from jax.experimental import pallas as pl
from jax.experimental.pallas import tpu as pltpu
from jax.experimental.pallas import tpu_sc as plsc

F32 = jnp.float32
V = 10000
VP = 10112
E = 320000
HID = 128
GD = 16
AD = 16
W = 128


def _sc_mesh():
    return plsc.VectorSubcoreMesh(core_axis_name="c", subcore_axis_name="s")


_SC_PARAMS = pltpu.CompilerParams(use_tc_tiling_on_sc=False)
_TC_PARAMS = pltpu.CompilerParams(dimension_semantics=("parallel",))


def _silu(t):
    return t * jax.nn.sigmoid(t)


def sc_gather_one(table, idx):
    nv, D = table.shape
    n = idx.shape[1]
    info = plsc.get_sparse_core_info()
    rps = nv // info.num_subcores

    @functools.partial(
        pl.kernel,
        out_type=jax.ShapeDtypeStruct((n, D), F32),
        mesh=_sc_mesh(),
        compiler_params=_SC_PARAMS,
        scratch_types=[pltpu.VMEM_SHARED((nv, D), F32)],
    )
    def k(tbl_hbm, i_hbm, o_hbm, tbl_s):
        sid = lax.axis_index("s")
        r0 = sid * rps
        pltpu.sync_copy(tbl_hbm.at[pl.ds(r0, rps)], tbl_s.at[pl.ds(r0, rps)])
        plsc.subcore_barrier()

        def body(i_v, o_v):
            pltpu.sync_copy(tbl_s.at[i_v.at[0]], o_v)

        pltpu.emit_pipeline(
            body,
            grid=(n // W,),
            in_specs=[pl.BlockSpec((1, W), lambda i: (0, i))],
            out_specs=[pl.BlockSpec((W, D), lambda i: (i, 0))],
            core_axis_name=("c", "s"),
            dimension_semantics=(pltpu.PARALLEL,),
        )(i_hbm, o_hbm)

    return k(table, idx)


def sc_scatter_add1(src, idx, zeros_tbl):
    n, D = src.shape
    nv = zeros_tbl.shape[0]
    info = plsc.get_sparse_core_info()
    nc, ns = info.num_cores, info.num_subcores
    rps = nv // ns

    @functools.partial(
        pl.kernel,
        out_type=jax.ShapeDtypeStruct((nc, nv, D), F32),
        mesh=_sc_mesh(),
        compiler_params=_SC_PARAMS,
        scratch_types=[pltpu.VMEM_SHARED((nv, D), F32)],
    )
    def k(s_hbm, i_hbm, z_hbm, o_hbm, acc):
        cid = lax.axis_index("c")
        sid = lax.axis_index("s")
        r0 = sid * rps
        pltpu.sync_copy(z_hbm.at[pl.ds(r0, rps)], acc.at[pl.ds(r0, rps)])
        plsc.subcore_barrier()

        def body(s_v, i_v):
            pltpu.sync_copy(s_v, acc.at[i_v.at[0]], add=True)

        pltpu.emit_pipeline(
            body,
            grid=(n // W,),
            in_specs=[
                pl.BlockSpec((W, D), lambda i: (i, 0)),
                pl.BlockSpec((1, W), lambda i: (0, i)),
            ],
            out_specs=[],
            core_axis_name=("c", "s"),
            dimension_semantics=(pltpu.PARALLEL,),
        )(s_hbm, i_hbm)
        plsc.subcore_barrier()
        pltpu.sync_copy(acc.at[pl.ds(r0, rps)], o_hbm.at[cid, pl.ds(r0, rps)])

    return k(src, idx, zeros_tbl)


def sc_scatter_add2(src_m, src_c, idx, zeros_m, zeros_c):
    n = src_m.shape[0]
    info = plsc.get_sparse_core_info()
    nc, ns = info.num_cores, info.num_subcores
    rps = VP // ns
    W2 = 64

    @functools.partial(
        pl.kernel,
        out_type=[jax.ShapeDtypeStruct((nc, VP, HID), F32),
                  jax.ShapeDtypeStruct((nc, VP, AD), F32)],
        mesh=_sc_mesh(),
        compiler_params=_SC_PARAMS,
        scratch_types=[pltpu.VMEM_SHARED((VP, HID), F32),
                       pltpu.VMEM_SHARED((VP, AD), F32),
                       pltpu.VMEM((W2, AD), F32)],
    )
    def k(sm_hbm, sc_hbm, i_hbm, zm_hbm, zc_hbm, om_hbm, oc_hbm, accm, accc,
          c_scr):
        cid = lax.axis_index("c")
        sid = lax.axis_index("s")
        r0 = sid * rps
        pltpu.sync_copy(zm_hbm.at[pl.ds(r0, rps)], accm.at[pl.ds(r0, rps)])
        pltpu.sync_copy(zc_hbm.at[pl.ds(r0, rps)], accc.at[pl.ds(r0, rps)])
        plsc.subcore_barrier()

        def body(sm_v, sc_v, i_v):
            pltpu.sync_copy(sm_v, accm.at[i_v.at[0]], add=True)

            @pl.loop(0, W2)
            def _(r):
                c_scr[r, :] = sc_v[r, pl.ds(0, AD)]

            pltpu.sync_copy(c_scr, accc.at[i_v.at[0]], add=True)

        pltpu.emit_pipeline(
            body,
            grid=(n // W2,),
            in_specs=[
                pl.BlockSpec((W2, HID), lambda i: (i, 0)),
                pl.BlockSpec((W2, HID), lambda i: (i, 0)),
                pl.BlockSpec((1, W2), lambda i: (0, i)),
            ],
            out_specs=[],
            core_axis_name=("c", "s"),
            dimension_semantics=(pltpu.PARALLEL,),
        )(sm_hbm, sc_hbm, i_hbm)
        plsc.subcore_barrier()
        pltpu.sync_copy(accm.at[pl.ds(r0, rps)], om_hbm.at[cid, pl.ds(r0, rps)])
        pltpu.sync_copy(accc.at[pl.ds(r0, rps)], oc_hbm.at[cid, pl.ds(r0, rps)])

    return k(src_m, src_c, idx, zeros_m, zeros_c)


def tc_edge1(H2, G2, We1h, We1c, we1d, be1, We2, be2, Wa1h, Wa1c, wa1d,
             ba1, Wa2, ba2):
    BE = 3200
    ne = H2.shape[0] // 2
    nb = ne // BE
    full = lambda a: pl.BlockSpec(a.shape, lambda i: (0,) * a.ndim)

    def body(hr_r, hc_r, gr_r, gc_r, we1h, we1c, we1d_, be1_, we2, be2_,
             wa1h, wa1c, wa1d_, ba1_, wa2, ba2_, m_o, au_o, p8_o):
        hr = hr_r[...]
        hc = hc_r[...]
        xr = gr_r[:, 0:3]
        xc = gc_r[:, 0:3]
        cr = gr_r[:, 3:7]
        diff = xr - xc
        dist_sq = jnp.sum(diff * diff, axis=1, keepdims=True)
        t = (jnp.dot(hr, we1h[...], preferred_element_type=F32)
             + jnp.dot(hc, we1c[...], preferred_element_type=F32)
             + dist_sq * we1d_[...] + be1_[...])
        m1 = _silu(t)
        t2 = jnp.dot(m1, we2[...], preferred_element_type=F32) + be2_[...]
        m_o[...] = _silu(t2)
        norm = jnp.sqrt(dist_sq)
        distf = jnp.maximum(norm, 1e-6)
        unit = (xc - xr) / distf
        ta = (jnp.dot(hr, wa1h[...], preferred_element_type=F32)
              + jnp.dot(cr, wa1c[...], preferred_element_type=F32)
              + distf * wa1d_[...] + ba1_[...])
        alpha = jnp.dot(_silu(ta), wa2[...], preferred_element_type=F32) + ba2_[...]
        au = alpha * unit
        au_o[...] = jnp.concatenate(
            [au, jnp.zeros((BE, HID - 3), F32)], axis=1)
        p8_o[...] = jnp.concatenate([unit, norm, cr], axis=1)

    return pl.pallas_call(
        body,
        grid=(nb,),
        compiler_params=_TC_PARAMS,
        in_specs=[
            pl.BlockSpec((BE, HID), lambda i: (i, 0)),
            pl.BlockSpec((BE, HID), lambda i: (i + nb, 0)),
            pl.BlockSpec((BE, HID), lambda i: (i, 0)),
            pl.BlockSpec((BE, HID), lambda i: (i + nb, 0)),
            full(We1h), full(We1c), full(we1d), full(be1), full(We2),
            full(be2), full(Wa1h), full(Wa1c), full(wa1d), full(ba1),
            full(Wa2), full(ba2),
        ],
        out_specs=[
            pl.BlockSpec((BE, HID), lambda i: (i, 0)),
            pl.BlockSpec((BE, HID), lambda i: (i, 0)),
            pl.BlockSpec((BE, 8), lambda i: (i, 0)),
        ],
        out_shape=[
            jax.ShapeDtypeStruct((ne, HID), F32),
            jax.ShapeDtypeStruct((ne, HID), F32),
            jax.ShapeDtypeStruct((ne, 8), F32),
        ],
    )(H2, H2, G2, G2, We1h, We1c, we1d, be1, We2, be2, Wa1h, Wa1c, wa1d,
      ba1, Wa2, ba2)


def tc_dnorm(dpart):
    def body(dp, dt_o):
        raw = jnp.sum(dp[:, :, 0:3], axis=0)
        nrm = jnp.sqrt(jnp.sum(raw * raw, axis=1, keepdims=True))
        d = raw / jnp.maximum(nrm, 1e-6)
        dt_o[...] = jnp.concatenate(
            [d, jnp.zeros((VP, HID - 3), F32)], axis=1)

    return pl.pallas_call(
        body,
        out_shape=jax.ShapeDtypeStruct((VP, HID), F32),
    )(dpart)


def tc_edge2(m, p8, Dr, ww1a, Ww1c, bw1, Ww2, bw2, Wc1, bc1, Wc2):
    BE = 3200
    ne = m.shape[0]
    nb = ne // BE
    full = lambda a: pl.BlockSpec(a.shape, lambda i: (0,) * a.ndim)

    def body(m_r, p8_r, dr_r, ww1a_, ww1c, bw1_, ww2, bw2_, wc1, bc1_, wc2,
             mw_o, c_o):
        unit = p8_r[:, 0:3]
        norm = p8_r[:, 3:4]
        cr = p8_r[:, 4:8]
        d_r = dr_r[:, 0:3]
        align = jnp.abs(jnp.sum(unit * d_r, axis=1, keepdims=True))
        tw = (align * ww1a_[...]
              + jnp.dot(cr, ww1c[...], preferred_element_type=F32) + bw1_[...])
        w = jax.nn.sigmoid(
            jnp.dot(_silu(tw), ww2[...], preferred_element_type=F32) + bw2_[...]
        ) + 0.5
        m_w = m_r[...] * w
        mw_o[...] = m_w
        tc1 = jnp.dot(m_w, wc1[...], preferred_element_type=F32) + bc1_[...]
        coordw = jnp.tanh(jnp.dot(_silu(tc1), wc2[...],
                                  preferred_element_type=F32))
        distf = jnp.maximum(norm, 1e-6)
        contrib = coordw * (-(unit * distf)) / (norm + 1.0)
        c_o[...] = jnp.concatenate(
            [contrib, jnp.zeros((BE, HID - 3), F32)], axis=1)

    return pl.pallas_call(
        body,
        grid=(nb,),
        compiler_params=_TC_PARAMS,
        in_specs=[
            pl.BlockSpec((BE, HID), lambda i: (i, 0)),
            pl.BlockSpec((BE, 8), lambda i: (i, 0)),
            pl.BlockSpec((BE, HID), lambda i: (i, 0)),
            full(ww1a), full(Ww1c), full(bw1), full(Ww2), full(bw2),
            full(Wc1), full(bc1), full(Wc2),
        ],
        out_specs=[
            pl.BlockSpec((BE, HID), lambda i: (i, 0)),
            pl.BlockSpec((BE, HID), lambda i: (i, 0)),
        ],
        out_shape=[
            jax.ShapeDtypeStruct((ne, HID), F32),
            jax.ShapeDtypeStruct((ne, HID), F32),
        ],
    )(m, p8, Dr, ww1a, Ww1c, bw1, Ww2, bw2, Wc1, bc1, Wc2)


def tc_node(mpart, cpart, h, x, Wn1h, Wn1m, bn1, Wn2, bn2, lng, lnb):
    BV = 2000
    nc = mpart.shape[0]
    full = lambda a: pl.BlockSpec(a.shape, lambda i: (0,) * a.ndim)

    def body(mp, cp, h_r, x_r, wn1h, wn1m, bn1_, wn2, bn2_, lng_, lnb_,
             ho_o, xn_o):
        msg = jnp.sum(mp[...], axis=0)
        coord = jnp.sum(cp[:, :, 0:3], axis=0)
        xn_o[...] = x_r[...] + coord
        t1 = (jnp.dot(h_r[...], wn1h[...], preferred_element_type=F32)
              + jnp.dot(msg, wn1m[...], preferred_element_type=F32)
              + bn1_[...])
        hmid = jnp.dot(_silu(t1), wn2[...], preferred_element_type=F32) + bn2_[...]
        y = h_r[...] + hmid
        mu = jnp.mean(y, axis=1, keepdims=True)
        var = jnp.mean((y - mu) ** 2, axis=1, keepdims=True)
        ho_o[...] = (y - mu) * lax.rsqrt(var + 1e-5) * lng_[...] + lnb_[...]

    return pl.pallas_call(
        body,
        grid=(V // BV,),
        compiler_params=_TC_PARAMS,
        in_specs=[
            pl.BlockSpec((nc, BV, HID), lambda i: (0, i, 0)),
            pl.BlockSpec((nc, BV, AD), lambda i: (0, i, 0)),
            pl.BlockSpec((BV, HID), lambda i: (i, 0)),
            pl.BlockSpec((BV, 3), lambda i: (i, 0)),
            full(Wn1h), full(Wn1m), full(bn1), full(Wn2), full(bn2),
            full(lng), full(lnb),
        ],
        out_specs=[
            pl.BlockSpec((BV, HID), lambda i: (i, 0)),
            pl.BlockSpec((BV, 3), lambda i: (i, 0)),
        ],
        out_shape=[
            jax.ShapeDtypeStruct((V, HID), F32),
            jax.ShapeDtypeStruct((V, 3), F32),
        ],
    )(mpart, cpart, h, x, Wn1h, Wn1m, bn1, Wn2, bn2, lng, lnb)


NCHUNK = 2
EC = E // NCHUNK


def kernel(h, x, curvature, edge_index, W_e1, b_e1, W_e2, b_e2, W_c1, b_c1,
           W_c2, W_n1, b_n1, W_n2, b_n2, ln_g, ln_b, W_a1, b_a1, W_a2, b_a2,
           W_w1, b_w1, W_w2, b_w2):
    row = edge_index[0].reshape(1, E)
    col = edge_index[1].reshape(1, E)
    h_tbl = jnp.concatenate([h, jnp.zeros((VP - V, HID), F32)], axis=0)
    g_tbl = jnp.concatenate(
        [jnp.concatenate([x, curvature, jnp.zeros((V, HID - 7), F32)], axis=1),
         jnp.zeros((VP - V, HID), F32)], axis=0)

    r2 = lambda a: a.reshape(1, -1)
    rows = [row[:, i * EC:(i + 1) * EC] for i in range(NCHUNK)]
    cols = [col[:, i * EC:(i + 1) * EC] for i in range(NCHUNK)]
    rcs = [jnp.concatenate([rows[i], cols[i]], axis=1) for i in range(NCHUNK)]

    H2 = [sc_gather_one(h_tbl, rcs[i]) for i in range(NCHUNK)]
    G2 = [sc_gather_one(g_tbl, rcs[i]) for i in range(NCHUNK)]

    e1 = [tc_edge1(
        H2[i], G2[i],
        W_e1[0:HID], W_e1[HID:2 * HID], r2(W_e1[2 * HID]), r2(b_e1),
        W_e2, r2(b_e2),
        W_a1[0:HID], W_a1[HID:HID + 4], r2(W_a1[HID + 4]), r2(b_a1),
        W_a2, r2(b_a2),
    ) for i in range(NCHUNK)]

    zh = jnp.zeros((VP, HID), F32)
    dparts = [sc_scatter_add1(e1[i][1], rows[i], zh) for i in range(NCHUNK)]
    dt = tc_dnorm(jnp.concatenate(dparts, axis=0))
    Dr = [sc_gather_one(dt, rows[i]) for i in range(NCHUNK)]

    e2 = [tc_edge2(
        e1[i][0], e1[i][2], Dr[i],
        r2(W_w1[0]), W_w1[1:5], r2(b_w1), W_w2, r2(b_w2),
        W_c1, r2(b_c1), W_c2,
    ) for i in range(NCHUNK)]

    zad = jnp.zeros((VP, AD), F32)
    parts = [sc_scatter_add2(e2[i][0], e2[i][1], cols[i], zh, zad)
             for i in range(NCHUNK)]
    mpart = jnp.concatenate([p[0] for p in parts], axis=0)
    cpart = jnp.concatenate([p[1] for p in parts], axis=0)
    h_out, x_new = tc_node(
        mpart, cpart, h, x,
        W_n1[0:HID], W_n1[HID:2 * HID], r2(b_n1), W_n2, r2(b_n2),
        r2(ln_g), r2(ln_b),
    )
    return (h_out, x_new)
